# baseline (speedup 1.0000x reference)
import functools

import jax
import jax.numpy as jnp
from jax.experimental import pallas as pl
from jax.experimental.pallas import tpu as pltpu

_EPS = 1e-5
_STAT_COLS = 16


def _compact_stats_kernel(x_ref, selw_ref, x2_ref, gram_ref, *, nb, cin, ho,
                          wo):
    wp = x2_ref.shape[-1]
    acc = [None] * 9
    for b in range(nb):
        xc = jnp.dot(x_ref[b].astype(jnp.bfloat16), selw_ref[...],
                     preferred_element_type=jnp.float32)
        x2_ref[b] = xc.reshape(cin, ho, wp).astype(jnp.bfloat16)
        ch = [xc[i * ho:(i + 1) * ho] for i in range(cin)]
        parts = [ch[i] for i in range(cin)]
        parts += [ch[i] * ch[k] for i in range(cin) for k in range(i, cin)]
        for r, t in enumerate(parts):
            s = jnp.sum(t)
            acc[r] = s if acc[r] is None else acc[r] + s

    row = jax.lax.broadcasted_iota(jnp.int32, gram_ref.shape[-2:], 0)
    z = jnp.zeros(gram_ref.shape[-2:], jnp.float32)
    for r, s in enumerate(acc):
        z = jnp.where(row == r, s, z)
    gram_ref[...] = z


def _conv_bn_relu_kernel(x_ref, gram_ref, m1_ref, m2_ref, w2_ref, gam_ref,
                         bet_ref, o_ref, *, nb, cin, cout):
    gsum = jnp.sum(gram_ref[...], axis=0)
    mean = jnp.dot(m1_ref[...], gsum, preferred_element_type=jnp.float32)
    ey2 = jnp.dot(m2_ref[...], gsum, preferred_element_type=jnp.float32)
    var = jnp.maximum(ey2 - mean * mean, 0.0)
    scale = gam_ref[...] * jax.lax.rsqrt(var + _EPS)
    shift = bet_ref[...] - mean * scale
    wf = (scale * w2_ref[...]).astype(jnp.bfloat16)
    for b in range(nb):
        y = jnp.dot(wf, x_ref[b * cin:(b + 1) * cin],
                    preferred_element_type=jnp.float32)
        o_ref[b] = jnp.maximum(y + shift, 0.0)


@jax.jit
def kernel(x_nchw, conv_w, conv_b, bn_gamma, bn_beta):
    n, cin, h, w = x_nchw.shape
    cout = conv_w.shape[0]
    ho, wo = (h + 1) // 2, (w + 1) // 2
    p = ho * wo
    del conv_b
    w2 = conv_w.reshape(cout, cin).astype(jnp.float32)

    xh = x_nchw[:, :, ::2, :].reshape(n, cin * ho, w)

    nb = next(d for d in (8, 4, 2, 1) if n % (2 * d) == 0)
    g1 = n // nb
    wp = ((wo + 127) // 128) * 128

    selw = ((jax.lax.broadcasted_iota(jnp.int32, (w, wp), 0) ==
             2 * jax.lax.broadcasted_iota(jnp.int32, (w, wp), 1)) &
            (jax.lax.broadcasted_iota(jnp.int32, (w, wp), 1) < wo)
            ).astype(jnp.bfloat16)

    x2c, gram = pl.pallas_call(
        functools.partial(_compact_stats_kernel, nb=nb, cin=cin, ho=ho,
                          wo=wo),
        out_shape=(jax.ShapeDtypeStruct((n, cin, ho, wp), jnp.bfloat16),
                   jax.ShapeDtypeStruct((g1, _STAT_COLS, 1), jnp.float32)),
        grid=(g1,),
        in_specs=[pl.BlockSpec((nb, cin * ho, w), lambda i: (i, 0, 0)),
                  pl.BlockSpec((w, wp), lambda i: (0, 0))],
        out_specs=(
            pl.BlockSpec((nb, cin, ho, wp), lambda i: (i, 0, 0, 0)),
            pl.BlockSpec((None, _STAT_COLS, 1), lambda i: (i, 0, 0)),
        ),
        compiler_params=pltpu.CompilerParams(
            dimension_semantics=("parallel",)),
        name="compact_stats",
    )(xh, selw)

    inv_count = 1.0 / float(n * p)
    n_pairs = (cin * (cin + 1)) // 2
    zpad = jnp.zeros((cout, _STAT_COLS - cin - n_pairs), jnp.float32)
    m1 = jnp.concatenate(
        [w2, jnp.zeros((cout, _STAT_COLS - cin), jnp.float32)],
        axis=1) * inv_count
    ii = [i for i in range(cin) for k in range(i, cin)]
    kk = [k for i in range(cin) for k in range(i, cin)]
    fac = jnp.array([1.0 if i == k else 2.0 for i, k in zip(ii, kk)],
                    jnp.float32)
    m2 = jnp.concatenate(
        [jnp.zeros((cout, cin), jnp.float32), w2[:, ii] * w2[:, kk] * fac,
         zpad], axis=1) * inv_count

    nb2 = nb
    g2 = n // nb2
    pp = ho * wp
    x2r = x2c.reshape(n * cin, pp)

    out3 = pl.pallas_call(
        functools.partial(_conv_bn_relu_kernel, nb=nb2, cin=cin, cout=cout),
        out_shape=jax.ShapeDtypeStruct((n, cout, pp), jnp.float32),
        grid=(g2,),
        in_specs=[
            pl.BlockSpec((nb2 * cin, pp), lambda i: (i, 0)),
            pl.BlockSpec((g1, _STAT_COLS, 1), lambda i: (0, 0, 0)),
            pl.BlockSpec((cout, _STAT_COLS), lambda i: (0, 0)),
            pl.BlockSpec((cout, _STAT_COLS), lambda i: (0, 0)),
            pl.BlockSpec((cout, cin), lambda i: (0, 0)),
            pl.BlockSpec((cout, 1), lambda i: (0, 0)),
            pl.BlockSpec((cout, 1), lambda i: (0, 0)),
        ],
        out_specs=pl.BlockSpec((nb2, cout, pp), lambda i: (i, 0, 0)),
        compiler_params=pltpu.CompilerParams(
            dimension_semantics=("parallel",)),
        name="folded_conv_bn_relu",
    )(x2r, gram, m1, m2, w2, bn_gamma[:, None], bn_beta[:, None])

    return out3.reshape(n, cout, ho, wp)[:, :, :, :wo]


# speedup vs baseline: 1.2956x; 1.2956x over previous
import functools

import jax
import jax.numpy as jnp
from jax.experimental import pallas as pl
from jax.experimental.pallas import tpu as pltpu

_EPS = 1e-5
_STAT_COLS = 16


def _compact_stats_kernel(x_ref, selw_ref, x2_ref, gram_ref, *, nb, cin, ho,
                          wo):
    acc = [None] * 9
    for b in range(nb):
        xc = jnp.dot(x_ref[b].astype(jnp.bfloat16), selw_ref[...],
                     preferred_element_type=jnp.float32)
        x2_ref[b] = xc.reshape(cin, ho, wo).astype(jnp.bfloat16)
        ch = [xc[i * ho:(i + 1) * ho] for i in range(cin)]
        parts = [ch[i] for i in range(cin)]
        parts += [ch[i] * ch[k] for i in range(cin) for k in range(i, cin)]
        for r, t in enumerate(parts):
            s = jnp.sum(t)
            acc[r] = s if acc[r] is None else acc[r] + s

    row = jax.lax.broadcasted_iota(jnp.int32, gram_ref.shape[-2:], 0)
    z = jnp.zeros(gram_ref.shape[-2:], jnp.float32)
    for r, s in enumerate(acc):
        z = jnp.where(row == r, s, z)
    gram_ref[...] = z


def _conv_bn_relu_kernel(x_ref, gram_ref, m1_ref, m2_ref, w2_ref, gam_ref,
                         bet_ref, o_ref, *, nb, cin, cout):
    gsum = jnp.sum(gram_ref[...], axis=0)
    mean = jnp.dot(m1_ref[...], gsum, preferred_element_type=jnp.float32)
    ey2 = jnp.dot(m2_ref[...], gsum, preferred_element_type=jnp.float32)
    var = jnp.maximum(ey2 - mean * mean, 0.0)
    scale = gam_ref[...] * jax.lax.rsqrt(var + _EPS)
    shift = bet_ref[...] - mean * scale
    wf = (scale * w2_ref[...]).astype(jnp.bfloat16)
    for b in range(nb):
        y = jnp.dot(wf, x_ref[b * cin:(b + 1) * cin],
                    preferred_element_type=jnp.float32)
        o_ref[b] = jnp.maximum(y + shift, 0.0)


@jax.jit
def kernel(x_nchw, conv_w, conv_b, bn_gamma, bn_beta):
    n, cin, h, w = x_nchw.shape
    cout = conv_w.shape[0]
    ho, wo = (h + 1) // 2, (w + 1) // 2
    p = ho * wo
    del conv_b
    w2 = conv_w.reshape(cout, cin).astype(jnp.float32)

    xh = x_nchw[:, :, ::2, :].reshape(n, cin * ho, w)

    nb = next(d for d in (8, 4, 2, 1) if n % (2 * d) == 0)
    g1 = n // nb
    selw = (jax.lax.broadcasted_iota(jnp.int32, (w, wo), 0) ==
            2 * jax.lax.broadcasted_iota(jnp.int32, (w, wo), 1)
            ).astype(jnp.bfloat16)

    x2c, gram = pl.pallas_call(
        functools.partial(_compact_stats_kernel, nb=nb, cin=cin, ho=ho,
                          wo=wo),
        out_shape=(jax.ShapeDtypeStruct((n, cin, ho, wo), jnp.bfloat16),
                   jax.ShapeDtypeStruct((g1, _STAT_COLS, 1), jnp.float32)),
        grid=(g1,),
        in_specs=[pl.BlockSpec((nb, cin * ho, w), lambda i: (i, 0, 0)),
                  pl.BlockSpec((w, wo), lambda i: (0, 0))],
        out_specs=(
            pl.BlockSpec((nb, cin, ho, wo), lambda i: (i, 0, 0, 0)),
            pl.BlockSpec((None, _STAT_COLS, 1), lambda i: (i, 0, 0)),
        ),
        compiler_params=pltpu.CompilerParams(
            dimension_semantics=("parallel",)),
        name="compact_stats",
    )(xh, selw)

    inv_count = 1.0 / float(n * p)
    n_pairs = (cin * (cin + 1)) // 2
    zpad = jnp.zeros((cout, _STAT_COLS - cin - n_pairs), jnp.float32)
    m1 = jnp.concatenate(
        [w2, jnp.zeros((cout, _STAT_COLS - cin), jnp.float32)],
        axis=1) * inv_count
    ii = [i for i in range(cin) for k in range(i, cin)]
    kk = [k for i in range(cin) for k in range(i, cin)]
    fac = jnp.array([1.0 if i == k else 2.0 for i, k in zip(ii, kk)],
                    jnp.float32)
    m2 = jnp.concatenate(
        [jnp.zeros((cout, cin), jnp.float32), w2[:, ii] * w2[:, kk] * fac,
         zpad], axis=1) * inv_count

    nb2 = nb
    g2 = n // nb2
    x2r = x2c.reshape(n * cin, p)

    out3 = pl.pallas_call(
        functools.partial(_conv_bn_relu_kernel, nb=nb2, cin=cin, cout=cout),
        out_shape=jax.ShapeDtypeStruct((n, cout, p), jnp.float32),
        grid=(g2,),
        in_specs=[
            pl.BlockSpec((nb2 * cin, p), lambda i: (i, 0)),
            pl.BlockSpec((g1, _STAT_COLS, 1), lambda i: (0, 0, 0)),
            pl.BlockSpec((cout, _STAT_COLS), lambda i: (0, 0)),
            pl.BlockSpec((cout, _STAT_COLS), lambda i: (0, 0)),
            pl.BlockSpec((cout, cin), lambda i: (0, 0)),
            pl.BlockSpec((cout, 1), lambda i: (0, 0)),
            pl.BlockSpec((cout, 1), lambda i: (0, 0)),
        ],
        out_specs=pl.BlockSpec((nb2, cout, p), lambda i: (i, 0, 0)),
        compiler_params=pltpu.CompilerParams(
            dimension_semantics=("parallel",)),
        name="folded_conv_bn_relu",
    )(x2r, gram, m1, m2, w2, bn_gamma[:, None], bn_beta[:, None])

    return out3.reshape(n, cout, ho, wo)


# speedup vs baseline: 1.3224x; 1.0206x over previous
import functools

import jax
import jax.numpy as jnp
from jax.experimental import pallas as pl
from jax.experimental.pallas import tpu as pltpu

_EPS = 1e-5
_STAT_COLS = 16


def _compact_stats_kernel(x_ref, selw_ref, x2_ref, gram_ref, *, nb, cin, ho,
                          wo):
    acc = [None] * 9
    for b in range(nb):
        xc = jnp.dot(x_ref[b].astype(jnp.bfloat16), selw_ref[...],
                     preferred_element_type=jnp.float32)
        x2_ref[b] = xc.reshape(cin, ho, wo).astype(jnp.bfloat16)
        ch = [xc[i * ho:(i + 1) * ho] for i in range(cin)]
        parts = [ch[i] for i in range(cin)]
        parts += [ch[i] * ch[k] for i in range(cin) for k in range(i, cin)]
        for r, t in enumerate(parts):
            s = jnp.sum(t)
            acc[r] = s if acc[r] is None else acc[r] + s

    row = jax.lax.broadcasted_iota(jnp.int32, gram_ref.shape[-2:], 0)
    z = jnp.zeros(gram_ref.shape[-2:], jnp.float32)
    for r, s in enumerate(acc):
        z = jnp.where(row == r, s, z)
    gram_ref[...] = z


def _conv_bn_relu_kernel(x_ref, gram_ref, m1_ref, m2_ref, w2_ref, gam_ref,
                         bet_ref, o_ref, *, nb, cin, cout):
    gsum = jnp.sum(gram_ref[...], axis=0)
    mean = jnp.dot(m1_ref[...], gsum, preferred_element_type=jnp.float32)
    ey2 = jnp.dot(m2_ref[...], gsum, preferred_element_type=jnp.float32)
    var = jnp.maximum(ey2 - mean * mean, 0.0)
    scale = gam_ref[...] * jax.lax.rsqrt(var + _EPS)
    shift = bet_ref[...] - mean * scale
    wf = (scale * w2_ref[...]).astype(jnp.bfloat16)
    for b in range(nb):
        y = jnp.dot(wf, x_ref[b * cin:(b + 1) * cin],
                    preferred_element_type=jnp.float32)
        o_ref[b] = jnp.maximum(y + shift, 0.0)


@jax.jit
def kernel(x_nchw, conv_w, conv_b, bn_gamma, bn_beta):
    n, cin, h, w = x_nchw.shape
    cout = conv_w.shape[0]
    ho, wo = (h + 1) // 2, (w + 1) // 2
    p = ho * wo
    del conv_b
    w2 = conv_w.reshape(cout, cin).astype(jnp.float32)

    xh = x_nchw[:, :, ::2, :].reshape(n, cin * ho, w)

    nb = next(d for d in (16, 8, 4, 2, 1) if n % (2 * d) == 0)
    g1 = n // nb
    selw = (jax.lax.broadcasted_iota(jnp.int32, (w, wo), 0) ==
            2 * jax.lax.broadcasted_iota(jnp.int32, (w, wo), 1)
            ).astype(jnp.bfloat16)

    x2c, gram = pl.pallas_call(
        functools.partial(_compact_stats_kernel, nb=nb, cin=cin, ho=ho,
                          wo=wo),
        out_shape=(jax.ShapeDtypeStruct((n, cin, ho, wo), jnp.bfloat16),
                   jax.ShapeDtypeStruct((g1, _STAT_COLS, 1), jnp.float32)),
        grid=(g1,),
        in_specs=[pl.BlockSpec((nb, cin * ho, w), lambda i: (i, 0, 0)),
                  pl.BlockSpec((w, wo), lambda i: (0, 0))],
        out_specs=(
            pl.BlockSpec((nb, cin, ho, wo), lambda i: (i, 0, 0, 0)),
            pl.BlockSpec((None, _STAT_COLS, 1), lambda i: (i, 0, 0)),
        ),
        compiler_params=pltpu.CompilerParams(
            dimension_semantics=("parallel",)),
        name="compact_stats",
    )(xh, selw)

    inv_count = 1.0 / float(n * p)
    n_pairs = (cin * (cin + 1)) // 2
    zpad = jnp.zeros((cout, _STAT_COLS - cin - n_pairs), jnp.float32)
    m1 = jnp.concatenate(
        [w2, jnp.zeros((cout, _STAT_COLS - cin), jnp.float32)],
        axis=1) * inv_count
    ii = [i for i in range(cin) for k in range(i, cin)]
    kk = [k for i in range(cin) for k in range(i, cin)]
    fac = jnp.array([1.0 if i == k else 2.0 for i, k in zip(ii, kk)],
                    jnp.float32)
    m2 = jnp.concatenate(
        [jnp.zeros((cout, cin), jnp.float32), w2[:, ii] * w2[:, kk] * fac,
         zpad], axis=1) * inv_count

    nb2 = nb
    g2 = n // nb2
    x2r = x2c.reshape(n * cin, p)

    out3 = pl.pallas_call(
        functools.partial(_conv_bn_relu_kernel, nb=nb2, cin=cin, cout=cout),
        out_shape=jax.ShapeDtypeStruct((n, cout, p), jnp.float32),
        grid=(g2,),
        in_specs=[
            pl.BlockSpec((nb2 * cin, p), lambda i: (i, 0)),
            pl.BlockSpec((g1, _STAT_COLS, 1), lambda i: (0, 0, 0)),
            pl.BlockSpec((cout, _STAT_COLS), lambda i: (0, 0)),
            pl.BlockSpec((cout, _STAT_COLS), lambda i: (0, 0)),
            pl.BlockSpec((cout, cin), lambda i: (0, 0)),
            pl.BlockSpec((cout, 1), lambda i: (0, 0)),
            pl.BlockSpec((cout, 1), lambda i: (0, 0)),
        ],
        out_specs=pl.BlockSpec((nb2, cout, p), lambda i: (i, 0, 0)),
        compiler_params=pltpu.CompilerParams(
            dimension_semantics=("parallel",),
            vmem_limit_bytes=48 * 1024 * 1024),
        name="folded_conv_bn_relu",
    )(x2r, gram, m1, m2, w2, bn_gamma[:, None], bn_beta[:, None])

    return out3.reshape(n, cout, ho, wo)


# speedup vs baseline: 1.4576x; 1.1023x over previous
import functools

import jax
import jax.numpy as jnp
from jax.experimental import pallas as pl
from jax.experimental.pallas import tpu as pltpu

_EPS = 1e-5
_STAT_COLS = 16


def _compact_stats_kernel(x_ref, selw_ref, x2_ref, gram_ref, *, nb, cin, ho,
                          wo):
    acc = [None] * 9
    for b in range(nb):
        xc = jnp.dot(x_ref[b].astype(jnp.bfloat16), selw_ref[...],
                     preferred_element_type=jnp.float32)
        x2_ref[b] = xc.reshape(cin, ho, wo).astype(jnp.bfloat16)
        ch = [xc[i * ho:(i + 1) * ho] for i in range(cin)]
        parts = [ch[i] for i in range(cin)]
        parts += [ch[i] * ch[k] for i in range(cin) for k in range(i, cin)]
        for r, t in enumerate(parts):
            s = jnp.sum(t)
            acc[r] = s if acc[r] is None else acc[r] + s

    row = jax.lax.broadcasted_iota(jnp.int32, gram_ref.shape[-2:], 0)
    z = jnp.zeros(gram_ref.shape[-2:], jnp.float32)
    for r, s in enumerate(acc):
        z = jnp.where(row == r, s, z)
    gram_ref[...] = z


def _conv_bn_relu_kernel(x_ref, gram_ref, m1_ref, m2_ref, w2_ref, gam_ref,
                         bet_ref, o_ref, *, nb, cin, cout):
    gsum = jnp.sum(gram_ref[...], axis=0)
    mean = jnp.dot(m1_ref[...], gsum, preferred_element_type=jnp.float32)
    ey2 = jnp.dot(m2_ref[...], gsum, preferred_element_type=jnp.float32)
    var = jnp.maximum(ey2 - mean * mean, 0.0)
    scale = gam_ref[...] * jax.lax.rsqrt(var + _EPS)
    shift = bet_ref[...] - mean * scale
    wf = (scale * w2_ref[...]).astype(jnp.bfloat16)
    for b in range(nb):
        y = jnp.dot(wf, x_ref[b * cin:(b + 1) * cin],
                    preferred_element_type=jnp.float32)
        o_ref[b] = jnp.maximum(y + shift, 0.0).astype(o_ref.dtype)


@jax.jit
def kernel(x_nchw, conv_w, conv_b, bn_gamma, bn_beta):
    n, cin, h, w = x_nchw.shape
    cout = conv_w.shape[0]
    ho, wo = (h + 1) // 2, (w + 1) // 2
    p = ho * wo
    del conv_b
    w2 = conv_w.reshape(cout, cin).astype(jnp.float32)

    xh = x_nchw[:, :, ::2, :].reshape(n, cin * ho, w)

    nb = next(d for d in (16, 8, 4, 2, 1) if n % (2 * d) == 0)
    g1 = n // nb
    selw = (jax.lax.broadcasted_iota(jnp.int32, (w, wo), 0) ==
            2 * jax.lax.broadcasted_iota(jnp.int32, (w, wo), 1)
            ).astype(jnp.bfloat16)

    x2c, gram = pl.pallas_call(
        functools.partial(_compact_stats_kernel, nb=nb, cin=cin, ho=ho,
                          wo=wo),
        out_shape=(jax.ShapeDtypeStruct((n, cin, ho, wo), jnp.bfloat16),
                   jax.ShapeDtypeStruct((g1, _STAT_COLS, 1), jnp.float32)),
        grid=(g1,),
        in_specs=[pl.BlockSpec((nb, cin * ho, w), lambda i: (i, 0, 0)),
                  pl.BlockSpec((w, wo), lambda i: (0, 0))],
        out_specs=(
            pl.BlockSpec((nb, cin, ho, wo), lambda i: (i, 0, 0, 0)),
            pl.BlockSpec((None, _STAT_COLS, 1), lambda i: (i, 0, 0)),
        ),
        compiler_params=pltpu.CompilerParams(
            dimension_semantics=("parallel",)),
        name="compact_stats",
    )(xh, selw)

    inv_count = 1.0 / float(n * p)
    n_pairs = (cin * (cin + 1)) // 2
    zpad = jnp.zeros((cout, _STAT_COLS - cin - n_pairs), jnp.float32)
    m1 = jnp.concatenate(
        [w2, jnp.zeros((cout, _STAT_COLS - cin), jnp.float32)],
        axis=1) * inv_count
    ii = [i for i in range(cin) for k in range(i, cin)]
    kk = [k for i in range(cin) for k in range(i, cin)]
    fac = jnp.array([1.0 if i == k else 2.0 for i, k in zip(ii, kk)],
                    jnp.float32)
    m2 = jnp.concatenate(
        [jnp.zeros((cout, cin), jnp.float32), w2[:, ii] * w2[:, kk] * fac,
         zpad], axis=1) * inv_count

    nb2 = nb
    g2 = n // nb2
    x2r = x2c.reshape(n * cin, p)

    out3 = pl.pallas_call(
        functools.partial(_conv_bn_relu_kernel, nb=nb2, cin=cin, cout=cout),
        out_shape=jax.ShapeDtypeStruct((n, cout, p), jnp.bfloat16),
        grid=(g2,),
        in_specs=[
            pl.BlockSpec((nb2 * cin, p), lambda i: (i, 0)),
            pl.BlockSpec((g1, _STAT_COLS, 1), lambda i: (0, 0, 0)),
            pl.BlockSpec((cout, _STAT_COLS), lambda i: (0, 0)),
            pl.BlockSpec((cout, _STAT_COLS), lambda i: (0, 0)),
            pl.BlockSpec((cout, cin), lambda i: (0, 0)),
            pl.BlockSpec((cout, 1), lambda i: (0, 0)),
            pl.BlockSpec((cout, 1), lambda i: (0, 0)),
        ],
        out_specs=pl.BlockSpec((nb2, cout, p), lambda i: (i, 0, 0)),
        compiler_params=pltpu.CompilerParams(
            dimension_semantics=("parallel",),
            vmem_limit_bytes=48 * 1024 * 1024),
        name="folded_conv_bn_relu",
    )(x2r, gram, m1, m2, w2, bn_gamma[:, None], bn_beta[:, None])

    return out3.reshape(n, cout, ho, wo).astype(jnp.float32)
